# baseline (speedup 1.0000x reference)
import functools

import jax
import jax.numpy as jnp
from jax import lax
from jax.experimental import pallas as pl
from jax.experimental.pallas import tpu as pltpu
from jax.experimental.pallas import tpu_sc as plsc

VOCAB = 1000000
EMB = 64
HID = 256
OUT = 2
BATCH = 4096
SEQ = 200
SEQ_P = 208

NC = 2
NS = 16
NW = NC * NS
B_PER_W = BATCH // NW
IDX_PER_W = B_PER_W * SEQ_P
CHUNK = SEQ_P // 2
N_BUF = 8
ROWS_PER_T = N_BUF // 2
T_ITERS = B_PER_W // ROWS_PER_T
NLANE = EMB // 16


@functools.partial(
    pl.kernel,
    out_type=jax.ShapeDtypeStruct((BATCH, EMB), jnp.float32),
    mesh=plsc.VectorSubcoreMesh(core_axis_name="c", subcore_axis_name="s"),
    scratch_types=[
        pltpu.VMEM((IDX_PER_W,), jnp.int32),
        pltpu.VMEM((B_PER_W, EMB), jnp.float32),
    ]
    + [pltpu.VMEM((CHUNK, EMB), jnp.float32) for _ in range(N_BUF)]
    + [pltpu.SemaphoreType.DMA for _ in range(N_BUF)],
    compiler_params=pltpu.CompilerParams(use_tc_tiling_on_sc=False),
)
def _pool(x_hbm, table_hbm, out_hbm, idx_v, pooled_v, *bufs_sems):
    gbufs = bufs_sems[:N_BUF]
    sems = bufs_sems[N_BUF:]
    wid = lax.axis_index("s") * NC + lax.axis_index("c")
    base = wid * IDX_PER_W

    pltpu.sync_copy(x_hbm.at[pl.ds(base, IDX_PER_W)], idx_v)

    def _issue(i, c):
        pltpu.async_copy(
            table_hbm.at[idx_v.at[pl.ds(i * CHUNK, CHUNK)]], gbufs[c], sems[c]
        )

    def _wait(i, c):
        pltpu.make_async_copy(
            table_hbm.at[idx_v.at[pl.ds(i * CHUNK, CHUNK)]], gbufs[c], sems[c]
        ).wait()

    def _accum(c, acc):
        def rbody(r, a):
            return tuple(
                a[k] + gbufs[c][r, pl.ds(k * 16, 16)] for k in range(NLANE)
            )

        return lax.fori_loop(0, CHUNK, rbody, acc, unroll=8)

    for c in range(N_BUF):
        _issue(jnp.int32(c), c)

    zeros = tuple(jnp.zeros((16,), jnp.float32) for _ in range(NLANE))

    def _super(t, carry, last):
        i0 = t * N_BUF
        for pair in range(ROWS_PER_T):
            acc = zeros
            for cc in range(2):
                c = pair * 2 + cc
                _wait(i0 + c, c)
                acc = _accum(c, acc)
                if not last:
                    _issue(i0 + c + N_BUF, c)
            row = t * ROWS_PER_T + pair
            for k in range(NLANE):
                pooled_v[row, pl.ds(k * 16, 16)] = acc[k]
        return carry

    lax.fori_loop(0, T_ITERS - 1, lambda t, cy: _super(t, cy, False), 0)
    _super(jnp.int32(T_ITERS - 1), 0, True)

    pltpu.sync_copy(pooled_v, out_hbm.at[pl.ds(wid * B_PER_W, B_PER_W)])


def _mlp_body(x_ref, w1_ref, b1_ref, w2_ref, b2_ref, o_ref):
    x = x_ref[...]
    h = jnp.dot(x, w1_ref[...], preferred_element_type=jnp.float32)
    h = jnp.maximum(h + b1_ref[...], 0.0)
    o_ref[...] = jnp.dot(h, w2_ref[...], preferred_element_type=jnp.float32) + b2_ref[...]


_OUT_PAD = 128
_MB = 512


def _mlp(pooled, W1, b1, W2p, b2p):
    return pl.pallas_call(
        _mlp_body,
        grid=(BATCH // _MB,),
        in_specs=[
            pl.BlockSpec((_MB, EMB), lambda i: (i, 0)),
            pl.BlockSpec((EMB, HID), lambda i: (0, 0)),
            pl.BlockSpec((1, HID), lambda i: (0, 0)),
            pl.BlockSpec((HID, _OUT_PAD), lambda i: (0, 0)),
            pl.BlockSpec((1, _OUT_PAD), lambda i: (0, 0)),
        ],
        out_specs=pl.BlockSpec((_MB, _OUT_PAD), lambda i: (i, 0)),
        out_shape=jax.ShapeDtypeStruct((BATCH, _OUT_PAD), jnp.float32),
    )(pooled, W1, b1, W2p, b2p)


def kernel(x_in, emb_table, W1, b1, W2, b2):
    x_pad = jnp.pad(x_in, ((0, 0), (0, SEQ_P - SEQ))).reshape(-1)
    pooled = _pool(x_pad, emb_table)
    W2p = jnp.zeros((HID, _OUT_PAD), jnp.float32).at[:, :OUT].set(W2)
    b2p = jnp.zeros((1, _OUT_PAD), jnp.float32).at[:, :OUT].set(b2)
    y = _mlp(pooled, W1, b1.reshape(1, HID), W2p, b2p)
    return y[:, :OUT]


# speedup vs baseline: 1.0022x; 1.0022x over previous
import functools

import jax
import jax.numpy as jnp
from jax import lax
from jax.experimental import pallas as pl
from jax.experimental.pallas import tpu as pltpu
from jax.experimental.pallas import tpu_sc as plsc

VOCAB = 1000000
EMB = 64
HID = 256
OUT = 2
BATCH = 4096
SEQ = 200
SEQ_P = 208

NC = 2
NS = 16
NW = NC * NS
B_PER_W = BATCH // NW
IDX_PER_W = B_PER_W * SEQ_P
CHUNK = SEQ_P // 2
N_BUF = 8
ROWS_PER_T = N_BUF // 2
T_ITERS = B_PER_W // ROWS_PER_T
NLANE = EMB // 16


@functools.partial(
    pl.kernel,
    out_type=jax.ShapeDtypeStruct((BATCH, EMB), jnp.float32),
    mesh=plsc.VectorSubcoreMesh(core_axis_name="c", subcore_axis_name="s"),
    scratch_types=[
        pltpu.VMEM((IDX_PER_W,), jnp.int32),
        pltpu.VMEM((B_PER_W, EMB), jnp.float32),
    ]
    + [pltpu.VMEM((CHUNK, EMB), jnp.float32) for _ in range(N_BUF)]
    + [pltpu.SemaphoreType.DMA for _ in range(N_BUF)],
    compiler_params=pltpu.CompilerParams(use_tc_tiling_on_sc=False),
)
def _pool(x_hbm, table_hbm, out_hbm, idx_v, pooled_v, *bufs_sems):
    gbufs = bufs_sems[:N_BUF]
    sems = bufs_sems[N_BUF:]
    wid = lax.axis_index("s") * NC + lax.axis_index("c")
    base = wid * IDX_PER_W

    pltpu.sync_copy(x_hbm.at[pl.ds(base, IDX_PER_W)], idx_v)

    def _issue(i, c):
        pltpu.async_copy(
            table_hbm.at[idx_v.at[pl.ds(i * CHUNK, CHUNK)]], gbufs[c], sems[c]
        )

    def _wait(i, c):
        pltpu.make_async_copy(
            table_hbm.at[idx_v.at[pl.ds(i * CHUNK, CHUNK)]], gbufs[c], sems[c]
        ).wait()

    NP = 4

    def _accum(c, acc):
        def rbody(i, a):
            r = i * NP
            return tuple(
                a[k * NP + p] + gbufs[c][r + p, pl.ds(k * 16, 16)]
                for k in range(NLANE)
                for p in range(NP)
            )

        return lax.fori_loop(0, CHUNK // NP, rbody, acc, unroll=2)

    for c in range(N_BUF):
        _issue(jnp.int32(c), c)

    zeros = tuple(jnp.zeros((16,), jnp.float32) for _ in range(NLANE * NP))

    def _super(t, carry, last):
        i0 = t * N_BUF
        for pair in range(ROWS_PER_T):
            acc = zeros
            for cc in range(2):
                c = pair * 2 + cc
                _wait(i0 + c, c)
                acc = _accum(c, acc)
                if not last:
                    _issue(i0 + c + N_BUF, c)
            row = t * ROWS_PER_T + pair
            for k in range(NLANE):
                s = (acc[k * NP] + acc[k * NP + 1]) + (
                    acc[k * NP + 2] + acc[k * NP + 3]
                )
                pooled_v[row, pl.ds(k * 16, 16)] = s
        return carry

    lax.fori_loop(0, T_ITERS - 1, lambda t, cy: _super(t, cy, False), 0)
    _super(jnp.int32(T_ITERS - 1), 0, True)

    pltpu.sync_copy(pooled_v, out_hbm.at[pl.ds(wid * B_PER_W, B_PER_W)])


def _mlp_body(x_ref, w1_ref, b1_ref, w2_ref, b2_ref, o_ref):
    x = x_ref[...]
    h = jnp.dot(x, w1_ref[...], preferred_element_type=jnp.float32)
    h = jnp.maximum(h + b1_ref[...], 0.0)
    o_ref[...] = jnp.dot(h, w2_ref[...], preferred_element_type=jnp.float32) + b2_ref[...]


_OUT_PAD = 128
_MB = 512


def _mlp(pooled, W1, b1, W2p, b2p):
    return pl.pallas_call(
        _mlp_body,
        grid=(BATCH // _MB,),
        in_specs=[
            pl.BlockSpec((_MB, EMB), lambda i: (i, 0)),
            pl.BlockSpec((EMB, HID), lambda i: (0, 0)),
            pl.BlockSpec((1, HID), lambda i: (0, 0)),
            pl.BlockSpec((HID, _OUT_PAD), lambda i: (0, 0)),
            pl.BlockSpec((1, _OUT_PAD), lambda i: (0, 0)),
        ],
        out_specs=pl.BlockSpec((_MB, _OUT_PAD), lambda i: (i, 0)),
        out_shape=jax.ShapeDtypeStruct((BATCH, _OUT_PAD), jnp.float32),
    )(pooled, W1, b1, W2p, b2p)


def kernel(x_in, emb_table, W1, b1, W2, b2):
    x_pad = jnp.pad(x_in, ((0, 0), (0, SEQ_P - SEQ))).reshape(-1)
    pooled = _pool(x_pad, emb_table)
    W2p = jnp.zeros((HID, _OUT_PAD), jnp.float32).at[:, :OUT].set(W2)
    b2p = jnp.zeros((1, _OUT_PAD), jnp.float32).at[:, :OUT].set(b2)
    y = _mlp(pooled, W1, b1.reshape(1, HID), W2p, b2p)
    return y[:, :OUT]


# speedup vs baseline: 1.0038x; 1.0015x over previous
import functools

import jax
import jax.numpy as jnp
from jax import lax
from jax.experimental import pallas as pl
from jax.experimental.pallas import tpu as pltpu
from jax.experimental.pallas import tpu_sc as plsc

VOCAB = 1000000
EMB = 64
HID = 256
OUT = 2
BATCH = 4096
SEQ = 200
SEQ_P = 208

NC = 2
NS = 16
NW = NC * NS
B_PER_W = BATCH // NW
IDX_PER_W = B_PER_W * SEQ_P
CHUNK = SEQ_P
N_BUF = 4
ROWS_PER_T = N_BUF
T_ITERS = B_PER_W // ROWS_PER_T
NLANE = EMB // 16


@functools.partial(
    pl.kernel,
    out_type=jax.ShapeDtypeStruct((BATCH, EMB), jnp.float32),
    mesh=plsc.VectorSubcoreMesh(core_axis_name="c", subcore_axis_name="s"),
    scratch_types=[
        pltpu.VMEM((IDX_PER_W,), jnp.int32),
        pltpu.VMEM((B_PER_W, EMB), jnp.float32),
    ]
    + [pltpu.VMEM((CHUNK, EMB), jnp.float32) for _ in range(N_BUF)]
    + [pltpu.SemaphoreType.DMA for _ in range(N_BUF)],
    compiler_params=pltpu.CompilerParams(use_tc_tiling_on_sc=False),
)
def _pool(x_hbm, table_hbm, out_hbm, idx_v, pooled_v, *bufs_sems):
    gbufs = bufs_sems[:N_BUF]
    sems = bufs_sems[N_BUF:]
    wid = lax.axis_index("s") * NC + lax.axis_index("c")
    base = wid * IDX_PER_W

    pltpu.sync_copy(x_hbm.at[pl.ds(base, IDX_PER_W)], idx_v)

    def _issue(i, c):
        pltpu.async_copy(
            table_hbm.at[idx_v.at[pl.ds(i * CHUNK, CHUNK)]], gbufs[c], sems[c]
        )

    def _wait(i, c):
        pltpu.make_async_copy(
            table_hbm.at[idx_v.at[pl.ds(i * CHUNK, CHUNK)]], gbufs[c], sems[c]
        ).wait()

    NP = 4

    def _accum(c, acc):
        def rbody(i, a):
            r = i * NP
            return tuple(
                a[k * NP + p] + gbufs[c][r + p, pl.ds(k * 16, 16)]
                for k in range(NLANE)
                for p in range(NP)
            )

        return lax.fori_loop(0, CHUNK // NP, rbody, acc, unroll=2)

    for c in range(N_BUF):
        _issue(jnp.int32(c), c)

    zeros = tuple(jnp.zeros((16,), jnp.float32) for _ in range(NLANE * NP))

    def _super(t, carry, last):
        i0 = t * N_BUF
        for c in range(N_BUF):
            _wait(i0 + c, c)
            acc = _accum(c, zeros)
            if not last:
                _issue(i0 + c + N_BUF, c)
            row = t * ROWS_PER_T + c
            for k in range(NLANE):
                s = (acc[k * NP] + acc[k * NP + 1]) + (
                    acc[k * NP + 2] + acc[k * NP + 3]
                )
                pooled_v[row, pl.ds(k * 16, 16)] = s
        return carry

    lax.fori_loop(0, T_ITERS - 1, lambda t, cy: _super(t, cy, False), 0)
    _super(jnp.int32(T_ITERS - 1), 0, True)

    pltpu.sync_copy(pooled_v, out_hbm.at[pl.ds(wid * B_PER_W, B_PER_W)])


def _mlp_body(x_ref, w1_ref, b1_ref, w2_ref, b2_ref, o_ref):
    x = x_ref[...]
    h = jnp.dot(x, w1_ref[...], preferred_element_type=jnp.float32)
    h = jnp.maximum(h + b1_ref[...], 0.0)
    o_ref[...] = jnp.dot(h, w2_ref[...], preferred_element_type=jnp.float32) + b2_ref[...]


_OUT_PAD = 128
_MB = 512


def _mlp(pooled, W1, b1, W2p, b2p):
    return pl.pallas_call(
        _mlp_body,
        grid=(BATCH // _MB,),
        in_specs=[
            pl.BlockSpec((_MB, EMB), lambda i: (i, 0)),
            pl.BlockSpec((EMB, HID), lambda i: (0, 0)),
            pl.BlockSpec((1, HID), lambda i: (0, 0)),
            pl.BlockSpec((HID, _OUT_PAD), lambda i: (0, 0)),
            pl.BlockSpec((1, _OUT_PAD), lambda i: (0, 0)),
        ],
        out_specs=pl.BlockSpec((_MB, _OUT_PAD), lambda i: (i, 0)),
        out_shape=jax.ShapeDtypeStruct((BATCH, _OUT_PAD), jnp.float32),
    )(pooled, W1, b1, W2p, b2p)


def kernel(x_in, emb_table, W1, b1, W2, b2):
    x_pad = jnp.pad(x_in, ((0, 0), (0, SEQ_P - SEQ))).reshape(-1)
    pooled = _pool(x_pad, emb_table)
    W2p = jnp.zeros((HID, _OUT_PAD), jnp.float32).at[:, :OUT].set(W2)
    b2p = jnp.zeros((1, _OUT_PAD), jnp.float32).at[:, :OUT].set(b2)
    y = _mlp(pooled, W1, b1.reshape(1, HID), W2p, b2p)
    return y[:, :OUT]


# speedup vs baseline: 1.5207x; 1.5150x over previous
import functools

import jax
import jax.numpy as jnp
import numpy as np
from jax import lax
from jax.experimental import pallas as pl
from jax.experimental.pallas import tpu as pltpu
from jax.experimental.pallas import tpu_sc as plsc

VOCAB = 1000000
EMB = 64
HID = 256
OUT = 2
BATCH = 4096
SEQ = 200

NC = 2
NS = 16
NW = NC * NS
B_PER_W = BATCH // NW
IDX_PER_W = B_PER_W * SEQ
CHUNK = 128
N_CHUNKS = IDX_PER_W // CHUNK

_DMAP = (
    (np.arange(IDX_PER_W) // SEQ).reshape(1, N_CHUNKS, CHUNK)
    + (np.arange(NS) * B_PER_W).reshape(NS, 1, 1)
).astype(np.int32)


@functools.partial(
    pl.kernel,
    out_type=jax.ShapeDtypeStruct((BATCH, EMB), jnp.float32),
    mesh=plsc.VectorSubcoreMesh(core_axis_name="c", subcore_axis_name="s"),
    scratch_types=[
        pltpu.VMEM((IDX_PER_W,), jnp.int32),
        pltpu.VMEM((N_CHUNKS, CHUNK), jnp.int32),
        pltpu.VMEM((CHUNK, EMB), jnp.float32),
        pltpu.VMEM_SHARED((NS * B_PER_W, EMB), jnp.float32),
        pltpu.SemaphoreType.DMA,
    ],
    compiler_params=pltpu.CompilerParams(use_tc_tiling_on_sc=False),
)
def _pool(x_hbm, dmap_hbm, table_hbm, out_hbm, idx_v, dmap_v, gbuf_v, pooled_sh, sem):
    sid = lax.axis_index("s")
    wid = sid * NC + lax.axis_index("c")
    base = wid * IDX_PER_W

    pltpu.sync_copy(x_hbm.at[pl.ds(base, IDX_PER_W)], idx_v)
    pltpu.sync_copy(dmap_hbm.at[sid], dmap_v)

    zero16 = jnp.zeros((16,), jnp.float32)

    def _zero(i, carry):
        for k in range(EMB // 16):
            gbuf_v[i, pl.ds(k * 16, 16)] = zero16
        return carry

    lax.fori_loop(0, CHUNK, _zero, 0, unroll=4)
    pltpu.sync_copy(gbuf_v, pooled_sh.at[pl.ds(sid * B_PER_W, B_PER_W)])

    def _chunk(j, carry):
        idx_slice = idx_v.at[pl.ds(j * CHUNK, CHUNK)]
        pltpu.async_copy(table_hbm.at[idx_slice], gbuf_v, sem).wait()
        pltpu.sync_copy(gbuf_v, pooled_sh.at[dmap_v.at[j]], add=True)
        return carry

    lax.fori_loop(0, N_CHUNKS, _chunk, 0)

    pltpu.sync_copy(
        pooled_sh.at[pl.ds(sid * B_PER_W, B_PER_W)],
        out_hbm.at[pl.ds(wid * B_PER_W, B_PER_W)],
    )


def _mlp_body(x_ref, w1_ref, b1_ref, w2_ref, b2_ref, o_ref):
    x = x_ref[...]
    h = jnp.dot(x, w1_ref[...], preferred_element_type=jnp.float32)
    h = jnp.maximum(h + b1_ref[...], 0.0)
    o_ref[...] = jnp.dot(h, w2_ref[...], preferred_element_type=jnp.float32) + b2_ref[...]


_OUT_PAD = 128
_MB = 512


def _mlp(pooled, W1, b1, W2p, b2p):
    return pl.pallas_call(
        _mlp_body,
        grid=(BATCH // _MB,),
        in_specs=[
            pl.BlockSpec((_MB, EMB), lambda i: (i, 0)),
            pl.BlockSpec((EMB, HID), lambda i: (0, 0)),
            pl.BlockSpec((1, HID), lambda i: (0, 0)),
            pl.BlockSpec((HID, _OUT_PAD), lambda i: (0, 0)),
            pl.BlockSpec((1, _OUT_PAD), lambda i: (0, 0)),
        ],
        out_specs=pl.BlockSpec((_MB, _OUT_PAD), lambda i: (i, 0)),
        out_shape=jax.ShapeDtypeStruct((BATCH, _OUT_PAD), jnp.float32),
    )(pooled, W1, b1, W2p, b2p)


def kernel(x_in, emb_table, W1, b1, W2, b2):
    x_flat = x_in.reshape(-1)
    emb_lin = lax.optimization_barrier(emb_table.reshape(-1))
    emb2 = emb_lin.reshape(VOCAB, EMB)
    pooled = _pool(x_flat, jnp.asarray(_DMAP), emb2)
    W2p = jnp.zeros((HID, _OUT_PAD), jnp.float32).at[:, :OUT].set(W2)
    b2p = jnp.zeros((1, _OUT_PAD), jnp.float32).at[:, :OUT].set(b2)
    y = _mlp(pooled, W1, b1.reshape(1, HID), W2p, b2p)
    return y[:, :OUT]


# speedup vs baseline: 1.6445x; 1.0814x over previous
import functools

import jax
import jax.numpy as jnp
import numpy as np
from jax import lax
from jax.experimental import pallas as pl
from jax.experimental.pallas import tpu as pltpu
from jax.experimental.pallas import tpu_sc as plsc

VOCAB = 1000000
EMB = 64
HID = 256
OUT = 2
BATCH = 4096
SEQ = 200

NC = 2
NS = 16
NW = NC * NS
B_PER_W = BATCH // NW
IDX_PER_W = B_PER_W * SEQ
CHUNK = 128
N_CHUNKS = IDX_PER_W // CHUNK

_DMAP = (
    (np.arange(IDX_PER_W) // SEQ).reshape(1, N_CHUNKS, CHUNK)
    + (np.arange(NS) * B_PER_W).reshape(NS, 1, 1)
).astype(np.int32)


@functools.partial(
    pl.kernel,
    out_type=jax.ShapeDtypeStruct((BATCH, EMB), jnp.float32),
    mesh=plsc.VectorSubcoreMesh(core_axis_name="c", subcore_axis_name="s"),
    scratch_types=[
        pltpu.VMEM((IDX_PER_W,), jnp.int32),
        pltpu.VMEM((N_CHUNKS, CHUNK), jnp.int32),
        pltpu.VMEM((4 * CHUNK, EMB), jnp.float32),
        pltpu.VMEM_SHARED((NS * B_PER_W, EMB), jnp.float32),
        pltpu.SemaphoreType.DMA,
    ],
    compiler_params=pltpu.CompilerParams(use_tc_tiling_on_sc=False),
)
def _pool(x_hbm, dmap_hbm, table_hbm, out_hbm, idx_v, dmap_v, gbuf_v, pooled_sh, sem):
    sid = lax.axis_index("s")
    wid = sid * NC + lax.axis_index("c")
    base = wid * IDX_PER_W

    pltpu.sync_copy(x_hbm.at[pl.ds(base, IDX_PER_W)], idx_v)
    pltpu.sync_copy(dmap_hbm.at[sid], dmap_v)

    zero16 = jnp.zeros((16,), jnp.float32)

    def _zero(i, carry):
        for k in range(EMB // 16):
            gbuf_v[i, pl.ds(k * 16, 16)] = zero16
        return carry

    lax.fori_loop(0, CHUNK, _zero, 0, unroll=4)
    pltpu.sync_copy(
        gbuf_v.at[pl.ds(0, CHUNK)], pooled_sh.at[pl.ds(sid * B_PER_W, B_PER_W)]
    )

    def _group(j, carry):
        idx_slice = idx_v.at[pl.ds(j * 4 * CHUNK, 4 * CHUNK)]
        pltpu.async_copy(table_hbm.at[idx_slice], gbuf_v, sem).wait()
        for t in range(4):
            pltpu.sync_copy(
                gbuf_v.at[pl.ds(t * CHUNK, CHUNK)],
                pooled_sh.at[dmap_v.at[4 * j + t]],
                add=True,
            )
        return carry

    lax.fori_loop(0, N_CHUNKS // 4, _group, 0)

    pltpu.sync_copy(
        pooled_sh.at[pl.ds(sid * B_PER_W, B_PER_W)],
        out_hbm.at[pl.ds(wid * B_PER_W, B_PER_W)],
    )


def _mlp_body(x_ref, w1_ref, b1_ref, w2_ref, b2_ref, o_ref):
    x = x_ref[...]
    h = jnp.dot(x, w1_ref[...], preferred_element_type=jnp.float32)
    h = jnp.maximum(h + b1_ref[...], 0.0)
    o_ref[...] = jnp.dot(h, w2_ref[...], preferred_element_type=jnp.float32) + b2_ref[...]


_OUT_PAD = 128
_MB = 512


def _mlp(pooled, W1, b1, W2p, b2p):
    return pl.pallas_call(
        _mlp_body,
        grid=(BATCH // _MB,),
        in_specs=[
            pl.BlockSpec((_MB, EMB), lambda i: (i, 0)),
            pl.BlockSpec((EMB, HID), lambda i: (0, 0)),
            pl.BlockSpec((1, HID), lambda i: (0, 0)),
            pl.BlockSpec((HID, _OUT_PAD), lambda i: (0, 0)),
            pl.BlockSpec((1, _OUT_PAD), lambda i: (0, 0)),
        ],
        out_specs=pl.BlockSpec((_MB, _OUT_PAD), lambda i: (i, 0)),
        out_shape=jax.ShapeDtypeStruct((BATCH, _OUT_PAD), jnp.float32),
    )(pooled, W1, b1, W2p, b2p)


def kernel(x_in, emb_table, W1, b1, W2, b2):
    x_flat = x_in.reshape(-1)
    emb_lin = lax.optimization_barrier(emb_table.reshape(-1))
    emb2 = emb_lin.reshape(VOCAB, EMB)
    pooled = _pool(x_flat, jnp.asarray(_DMAP), emb2)
    W2p = jnp.zeros((HID, _OUT_PAD), jnp.float32).at[:, :OUT].set(W2)
    b2p = jnp.zeros((1, _OUT_PAD), jnp.float32).at[:, :OUT].set(b2)
    y = _mlp(pooled, W1, b1.reshape(1, HID), W2p, b2p)
    return y[:, :OUT]
